# Initial kernel scaffold; baseline (speedup 1.0000x reference)
import jax
import jax.numpy as jnp
from jax.experimental import pallas as pl


def kernel(x, edge_index, W_lin0, W_gcn0, gamma0, beta0, W_lin1, W_gcn1, gamma1, beta1):
    raise NotImplementedError("write your pallas kernel here")


# speedup vs baseline: 10.4054x; 10.4054x over previous
import functools

import jax
import jax.numpy as jnp
from jax import lax
from jax.experimental import pallas as pl
from jax.experimental.pallas import tpu as pltpu
from jax.experimental.pallas import tpu_sc as plsc

N = 10000
N2 = 10240
E = 320000
D = 128
EPS = 1e-5
NC = 2
NS = 16
NW = NC * NS
EPW = E // NW
CH = 80
NCHUNK = EPW // CH
RPT = N2 // NS
RPT2 = N2 // NS


def _sc_mesh():
    return plsc.VectorSubcoreMesh(core_axis_name="c", subcore_axis_name="s")


def _deg_call(col, zeros_n1, ones_ch):
    dw = ones_ch.shape[1]
    @functools.partial(
        pl.kernel,
        out_type=jax.ShapeDtypeStruct((NC, N2, dw), jnp.float32),
        mesh=_sc_mesh(),
        scratch_types=[
            pltpu.VMEM((CH,), jnp.int32),
            pltpu.VMEM((CH, dw), jnp.float32),
            pltpu.VMEM_SHARED((N2, dw), jnp.float32),
        ],
    )
    def deg_kernel(col_hbm, zeros_hbm, ones_hbm, out_hbm, cidx_v, ones_v, deg_sh):
        c = lax.axis_index("c")
        s = lax.axis_index("s")
        wid = s * NC + c
        pltpu.sync_copy(ones_hbm, ones_v)
        pltpu.sync_copy(zeros_hbm.at[pl.ds(RPT2 * s, RPT2)],
                        deg_sh.at[pl.ds(RPT2 * s, RPT2)])
        plsc.subcore_barrier()

        def step(g, carry):
            eb = pl.multiple_of(wid * EPW + g * CH, 8)
            pltpu.sync_copy(col_hbm.at[pl.ds(eb, CH)], cidx_v)
            pltpu.sync_copy(ones_v, deg_sh.at[cidx_v], add=True)
            return carry

        lax.fori_loop(0, NCHUNK, step, 0)
        plsc.subcore_barrier()
        pltpu.sync_copy(deg_sh.at[pl.ds(RPT2 * s, RPT2)],
                        out_hbm.at[c, pl.ds(RPT2 * s, RPT2)])

    return deg_kernel(col, zeros_n1, ones_ch)


def _edge_call(hp, row, col, zeros_nd):
    @functools.partial(
        pl.kernel,
        out_type=jax.ShapeDtypeStruct((NC, N2, D), jnp.float32),
        mesh=_sc_mesh(),
        scratch_types=[
            pltpu.VMEM((CH,), jnp.int32),
            pltpu.VMEM((CH,), jnp.int32),
            pltpu.VMEM((CH, D), jnp.float32),
            pltpu.VMEM_SHARED((N2, D), jnp.float32),
            pltpu.SemaphoreType.DMA,
        ],
    )
    def edge_kernel(hp_hbm, row_hbm, col_hbm, zeros_hbm, out_hbm,
                    ridx_v, cidx_v, rows_v, acc_sh, gsem):
        c = lax.axis_index("c")
        s = lax.axis_index("s")
        wid = s * NC + c
        pltpu.sync_copy(zeros_hbm.at[pl.ds(RPT * s, RPT)],
                        acc_sh.at[pl.ds(RPT * s, RPT)])
        plsc.subcore_barrier()

        def step(g, carry):
            eb = pl.multiple_of(wid * EPW + g * CH, 8)
            pltpu.sync_copy(row_hbm.at[pl.ds(eb, CH)], ridx_v)
            pltpu.sync_copy(col_hbm.at[pl.ds(eb, CH)], cidx_v)
            pltpu.async_copy(hp_hbm.at[ridx_v], rows_v, gsem).wait()
            pltpu.sync_copy(rows_v, acc_sh.at[cidx_v], add=True)
            return carry

        lax.fori_loop(0, NCHUNK, step, 0)
        plsc.subcore_barrier()
        pltpu.sync_copy(acc_sh.at[pl.ds(RPT * s, RPT)],
                        out_hbm.at[c, pl.ds(RPT * s, RPT)])

    return edge_kernel(hp, row, col, zeros_nd)


def _dis_from(degp_ref):
    deg = (degp_ref[0] + degp_ref[1])[:N, 0:1]
    return jnp.where(deg > 0.0, lax.rsqrt(deg), 0.0)


def _bn(y, g_ref, b_ref):
    mean = jnp.mean(y, axis=0, keepdims=True)
    var = jnp.mean((y - mean) ** 2, axis=0, keepdims=True)
    return (y - mean) * lax.rsqrt(var + EPS) * g_ref[...][None, :] + b_ref[...][None, :]


def _prep_body(h_ref, wg_ref, wl_ref, degp_ref, hp_ref, hl_ref):
    dis = _dis_from(degp_ref)
    h = h_ref[...]
    hp_ref[...] = jnp.dot(h, wg_ref[...], preferred_element_type=jnp.float32) * dis
    hl_ref[...] = jnp.dot(h, wl_ref[...], preferred_element_type=jnp.float32)


def _prep_call(h, wg, wl, degp):
    return pl.pallas_call(
        _prep_body,
        out_shape=(jax.ShapeDtypeStruct((N, D), jnp.float32),
                   jax.ShapeDtypeStruct((N, D), jnp.float32)),
    )(h, wg, wl, degp)


def _mid_body(hl_ref, acc_ref, degp_ref, g_ref, b_ref, wg_ref, wl_ref,
              hp_ref, hlo_ref):
    dis = _dis_from(degp_ref)
    y = hl_ref[...] + dis * (acc_ref[0] + acc_ref[1])[:N]
    h = jnp.maximum(_bn(y, g_ref, b_ref), 0.0)
    hp_ref[...] = jnp.dot(h, wg_ref[...], preferred_element_type=jnp.float32) * dis
    hlo_ref[...] = jnp.dot(h, wl_ref[...], preferred_element_type=jnp.float32)


def _mid_call(hl, acc, degp, gamma, beta, wg, wl):
    return pl.pallas_call(
        _mid_body,
        out_shape=(jax.ShapeDtypeStruct((N, D), jnp.float32),
                   jax.ShapeDtypeStruct((N, D), jnp.float32)),
    )(hl, acc, degp, gamma, beta, wg, wl)


def _fin_body(hl_ref, acc_ref, degp_ref, g_ref, b_ref, out_ref):
    dis = _dis_from(degp_ref)
    y = hl_ref[...] + dis * (acc_ref[0] + acc_ref[1])[:N]
    out_ref[...] = _bn(y, g_ref, b_ref)


def _fin_call(hl, acc, degp, gamma, beta):
    return pl.pallas_call(
        _fin_body,
        out_shape=jax.ShapeDtypeStruct((N, D), jnp.float32),
    )(hl, acc, degp, gamma, beta)


def kernel(x, edge_index, W_lin0, W_gcn0, gamma0, beta0,
           W_lin1, W_gcn1, gamma1, beta1):
    row = edge_index[0]
    col = edge_index[1]
    zeros_nd = jnp.zeros((N2, D), jnp.float32)
    zeros_n1 = jnp.zeros((N2, 128), jnp.float32)
    ones_ch = jnp.ones((CH, 128), jnp.float32)

    degp = _deg_call(col, zeros_n1, ones_ch)
    hp0, hl0 = _prep_call(x, W_gcn0, W_lin0, degp)
    acc0 = _edge_call(hp0, row, col, zeros_nd)
    hp1, hl1 = _mid_call(hl0, acc0, degp, gamma0, beta0, W_gcn1, W_lin1)
    acc1 = _edge_call(hp1, row, col, zeros_nd)
    return _fin_call(hl1, acc1, degp, gamma1, beta1)


# speedup vs baseline: 15.0474x; 1.4461x over previous
import functools

import jax
import jax.numpy as jnp
from jax import lax
from jax.experimental import pallas as pl
from jax.experimental.pallas import tpu as pltpu
from jax.experimental.pallas import tpu_sc as plsc

N = 10000
N2 = 10240
E = 320000
D = 128
EPS = 1e-5
NC = 2
NS = 16
NW = NC * NS
EPW = E // NW
CH = 80
NCHUNK = EPW // CH
RPT = N2 // NS
RPT2 = N2 // NS


def _sc_mesh():
    return plsc.VectorSubcoreMesh(core_axis_name="c", subcore_axis_name="s")


def _deg_call(col, zeros_n1, ones_ch):
    dw = ones_ch.shape[1]
    @functools.partial(
        pl.kernel,
        out_type=jax.ShapeDtypeStruct((NC, N2, dw), jnp.float32),
        mesh=_sc_mesh(),
        scratch_types=[
            pltpu.VMEM((CH,), jnp.int32),
            pltpu.VMEM((CH, dw), jnp.float32),
            pltpu.VMEM_SHARED((N2, dw), jnp.float32),
        ],
    )
    def deg_kernel(col_hbm, zeros_hbm, ones_hbm, out_hbm, cidx_v, ones_v, deg_sh):
        c = lax.axis_index("c")
        s = lax.axis_index("s")
        wid = s * NC + c
        pltpu.sync_copy(ones_hbm, ones_v)
        pltpu.sync_copy(zeros_hbm.at[pl.ds(RPT2 * s, RPT2)],
                        deg_sh.at[pl.ds(RPT2 * s, RPT2)])
        plsc.subcore_barrier()

        def step(g, carry):
            eb = pl.multiple_of(wid * EPW + g * CH, 8)
            pltpu.sync_copy(col_hbm.at[pl.ds(eb, CH)], cidx_v)
            pltpu.sync_copy(ones_v, deg_sh.at[cidx_v], add=True)
            return carry

        lax.fori_loop(0, NCHUNK, step, 0)
        plsc.subcore_barrier()
        pltpu.sync_copy(deg_sh.at[pl.ds(RPT2 * s, RPT2)],
                        out_hbm.at[c, pl.ds(RPT2 * s, RPT2)])

    return deg_kernel(col, zeros_n1, ones_ch)


def _edge_call(hp, row, col, zeros_nd):
    @functools.partial(
        pl.kernel,
        out_type=jax.ShapeDtypeStruct((NC, N2, D), jnp.float32),
        mesh=_sc_mesh(),
        scratch_types=[
            pltpu.VMEM((CH,), jnp.int32),
            pltpu.VMEM((CH,), jnp.int32),
            pltpu.VMEM((CH,), jnp.int32),
            pltpu.VMEM((CH,), jnp.int32),
            pltpu.VMEM((CH, D), jnp.float32),
            pltpu.VMEM((CH, D), jnp.float32),
            pltpu.VMEM_SHARED((N2, D), jnp.float32),
            pltpu.SemaphoreType.DMA,
            pltpu.SemaphoreType.DMA,
        ],
    )
    def edge_kernel(hp_hbm, row_hbm, col_hbm, zeros_hbm, out_hbm,
                    ridx0, cidx0, ridx1, cidx1, rows0, rows1, acc_sh,
                    sem0, sem1):
        c = lax.axis_index("c")
        s = lax.axis_index("s")
        wid = s * NC + c
        base = wid * EPW
        pltpu.sync_copy(zeros_hbm.at[pl.ds(RPT * s, RPT)],
                        acc_sh.at[pl.ds(RPT * s, RPT)])
        plsc.subcore_barrier()

        def fetch(g, ridx, cidx, rows, sem):
            eb = pl.multiple_of(base + g * CH, 8)
            pltpu.sync_copy(row_hbm.at[pl.ds(eb, CH)], ridx)
            pltpu.sync_copy(col_hbm.at[pl.ds(eb, CH)], cidx)
            pltpu.async_copy(hp_hbm.at[ridx], rows, sem)

        def drain(ridx, cidx, rows, sem):
            pltpu.make_async_copy(hp_hbm.at[ridx], rows, sem).wait()
            pltpu.sync_copy(rows, acc_sh.at[cidx], add=True)

        fetch(0, ridx0, cidx0, rows0, sem0)

        def body(k, carry):
            g0 = 2 * k
            fetch(g0 + 1, ridx1, cidx1, rows1, sem1)
            drain(ridx0, cidx0, rows0, sem0)

            @pl.when(g0 + 2 < NCHUNK)
            def _():
                fetch(g0 + 2, ridx0, cidx0, rows0, sem0)

            drain(ridx1, cidx1, rows1, sem1)
            return carry

        lax.fori_loop(0, NCHUNK // 2, body, 0)
        if NCHUNK % 2 == 1:
            drain(ridx0, cidx0, rows0, sem0)
        plsc.subcore_barrier()
        pltpu.sync_copy(acc_sh.at[pl.ds(RPT * s, RPT)],
                        out_hbm.at[c, pl.ds(RPT * s, RPT)])

    return edge_kernel(hp, row, col, zeros_nd)


def _dis_from(degp_ref):
    deg = (degp_ref[0] + degp_ref[1])[:N, 0:1]
    return jnp.where(deg > 0.0, lax.rsqrt(deg), 0.0)


def _bn(y, g_ref, b_ref):
    mean = jnp.mean(y, axis=0, keepdims=True)
    var = jnp.mean((y - mean) ** 2, axis=0, keepdims=True)
    return (y - mean) * lax.rsqrt(var + EPS) * g_ref[...][None, :] + b_ref[...][None, :]


def _prep_body(h_ref, wg_ref, wl_ref, degp_ref, hp_ref, hl_ref):
    dis = _dis_from(degp_ref)
    h = h_ref[...]
    hp_ref[...] = jnp.dot(h, wg_ref[...], preferred_element_type=jnp.float32) * dis
    hl_ref[...] = jnp.dot(h, wl_ref[...], preferred_element_type=jnp.float32)


def _prep_call(h, wg, wl, degp):
    return pl.pallas_call(
        _prep_body,
        out_shape=(jax.ShapeDtypeStruct((N, D), jnp.float32),
                   jax.ShapeDtypeStruct((N, D), jnp.float32)),
    )(h, wg, wl, degp)


def _mid_body(hl_ref, acc_ref, degp_ref, g_ref, b_ref, wg_ref, wl_ref,
              hp_ref, hlo_ref):
    dis = _dis_from(degp_ref)
    y = hl_ref[...] + dis * (acc_ref[0] + acc_ref[1])[:N]
    h = jnp.maximum(_bn(y, g_ref, b_ref), 0.0)
    hp_ref[...] = jnp.dot(h, wg_ref[...], preferred_element_type=jnp.float32) * dis
    hlo_ref[...] = jnp.dot(h, wl_ref[...], preferred_element_type=jnp.float32)


def _mid_call(hl, acc, degp, gamma, beta, wg, wl):
    return pl.pallas_call(
        _mid_body,
        out_shape=(jax.ShapeDtypeStruct((N, D), jnp.float32),
                   jax.ShapeDtypeStruct((N, D), jnp.float32)),
    )(hl, acc, degp, gamma, beta, wg, wl)


def _fin_body(hl_ref, acc_ref, degp_ref, g_ref, b_ref, out_ref):
    dis = _dis_from(degp_ref)
    y = hl_ref[...] + dis * (acc_ref[0] + acc_ref[1])[:N]
    out_ref[...] = _bn(y, g_ref, b_ref)


def _fin_call(hl, acc, degp, gamma, beta):
    return pl.pallas_call(
        _fin_body,
        out_shape=jax.ShapeDtypeStruct((N, D), jnp.float32),
    )(hl, acc, degp, gamma, beta)


def kernel(x, edge_index, W_lin0, W_gcn0, gamma0, beta0,
           W_lin1, W_gcn1, gamma1, beta1):
    row = edge_index[0]
    col = edge_index[1]
    zeros_nd = jnp.zeros((N2, D), jnp.float32)
    zeros_n1 = jnp.zeros((N2, 128), jnp.float32)
    ones_ch = jnp.ones((CH, 128), jnp.float32)

    degp = _deg_call(col, zeros_n1, ones_ch)
    hp0, hl0 = _prep_call(x, W_gcn0, W_lin0, degp)
    acc0 = _edge_call(hp0, row, col, zeros_nd)
    hp1, hl1 = _mid_call(hl0, acc0, degp, gamma0, beta0, W_gcn1, W_lin1)
    acc1 = _edge_call(hp1, row, col, zeros_nd)
    return _fin_call(hl1, acc1, degp, gamma1, beta1)
